# baseline (speedup 1.0000x reference)
import functools

import jax
import jax.numpy as jnp
from jax import lax
from jax.experimental import pallas as pl
from jax.experimental.pallas import tpu as pltpu
from jax.experimental.pallas import tpu_sc as plsc

_BINS = 10
_NS = 16
_NC = 2


def _allsum(v):
    return jnp.zeros((16,), v.dtype) + jnp.sum(v)


def _lse_kernel(x_ref, out_ref):
    xb = x_ref[...]
    m = jnp.max(xb, axis=2)
    se = jnp.sum(jnp.exp(xb - m[:, :, None]), axis=2)
    out_ref[...] = m + jnp.log(se)


def _sc_gather_body(xflat_hbm, tgt_hbm, xt_hbm, t_v, idx_v, xt_v, sem,
                    *, n_rows, n_cols):
    wid = lax.axis_index("s") * _NC + lax.axis_index("c")
    chunk = n_rows // (_NS * _NC)
    base = pl.multiple_of(wid * chunk, 8)
    pltpu.sync_copy(tgt_hbm.at[pl.ds(base, chunk)], t_v)

    lane = lax.iota(jnp.int32, 16)
    nrow = chunk // 128
    for j in range(nrow):
        for k in range(8):
            off = j * 128 + k * 16
            t16 = t_v[pl.ds(off, 16)]
            rows16 = lane + (base + off)
            idx_v[j, pl.ds(k * 16, 16)] = rows16 * n_cols + t16
    copies = []
    for j in range(nrow):
        copies.append(pltpu.async_copy(
            xflat_hbm.at[idx_v.at[j]], xt_v.at[pl.ds(j * 128, 128)], sem))
    for cp in copies:
        cp.wait()
    pltpu.sync_copy(xt_v, xt_hbm.at[pl.ds(base, chunk)])


def _sc_hist_body(lse_hbm, xt_hbm, out_hbm, lse_v, xt_v, both_v, gather_v,
                  out_v, shared_both, *, n_rows):
    sid = lax.axis_index("s")
    chunk = n_rows // _NS
    base = pl.multiple_of(sid * chunk, 8)
    pltpu.sync_copy(lse_hbm.at[pl.ds(base, chunk)], lse_v)
    pltpu.sync_copy(xt_hbm.at[pl.ds(base, chunk)], xt_v)

    zeros = jnp.zeros((16,), jnp.float32)

    def step(j, carry):
        cnts, sums = carry
        ce16 = lse_v[pl.ds(j * 16, 16)] - xt_v[pl.ds(j * 16, 16)]
        g = jnp.abs(jnp.exp(-ce16) - 1.0)
        b16 = (g * (_BINS - 0.0001)).astype(jnp.int32)
        new_c = []
        new_s = []
        for b in range(_BINS):
            m = b16 == b
            new_c.append(cnts[b] + jnp.where(m, 1.0, 0.0))
            new_s.append(sums[b] + jnp.where(m, ce16, 0.0))
        return new_c, new_s

    cnts, sums = lax.fori_loop(
        0, chunk // 16, step, ([zeros] * _BINS, [zeros] * _BINS))

    lane = lax.iota(jnp.int32, 16)
    hist = zeros
    ssum = zeros
    for b in range(_BINS):
        hist = jnp.where(lane == b, _allsum(cnts[b]), hist)
        ssum = jnp.where(lane == b, _allsum(sums[b]), ssum)
    both_v[pl.ds(0, 16)] = hist
    both_v[pl.ds(16, 16)] = ssum

    pltpu.sync_copy(both_v, shared_both.at[pl.ds(pl.multiple_of(sid * 32, 8), 32)])
    plsc.subcore_barrier()

    @pl.when(sid == 0)
    def _finish():
        pltpu.sync_copy(shared_both, gather_v)
        cnt = jnp.zeros((16,), jnp.float32)
        ssum2 = jnp.zeros((16,), jnp.float32)
        for w in range(_NS):
            cnt = cnt + gather_v[pl.ds(w * 32, 16)]
            ssum2 = ssum2 + gather_v[pl.ds(w * 32 + 16, 16)]
        valid = lane < _BINS
        nonempty = _allsum(jnp.where(valid & (cnt > 0.0), 1.0, 0.0))
        gd = jnp.maximum(cnt * nonempty, 0.0001)
        loss = _allsum(jnp.where(valid, ssum2 / gd, 0.0)) * (1.0 / n_rows)
        out_v[...] = loss
        pltpu.sync_copy(out_v, out_hbm)


def kernel(x, target):
    n, c = x.shape
    block_rows = 4096
    nblocks = n // block_rows
    sb = block_rows // 128
    x3 = x.reshape(n // 128, 128, c)
    wchunk = n // (_NS * _NC)

    gather_mesh = plsc.VectorSubcoreMesh(
        core_axis_name="c", subcore_axis_name="s", num_cores=_NC)
    sc_gather = functools.partial(
        pl.kernel,
        out_type=jax.ShapeDtypeStruct((n,), jnp.float32),
        mesh=gather_mesh,
        compiler_params=pltpu.CompilerParams(needs_layout_passes=False),
        scratch_types=[
            pltpu.VMEM((wchunk,), jnp.int32),
            pltpu.VMEM((wchunk // 128, 128), jnp.int32),
            pltpu.VMEM((wchunk,), jnp.float32),
            pltpu.SemaphoreType.DMA,
        ],
    )(functools.partial(_sc_gather_body, n_rows=n, n_cols=c))
    xt = sc_gather(x.reshape(n * c), target)

    lse = pl.pallas_call(
        _lse_kernel,
        grid=(nblocks,),
        in_specs=[pl.BlockSpec((sb, 128, c), lambda i: (i, 0, 0))],
        out_specs=pl.BlockSpec((sb, 128), lambda i: (i, 0)),
        out_shape=jax.ShapeDtypeStruct((n // 128, 128), jnp.float32),
    )(x3)

    hist_mesh = plsc.VectorSubcoreMesh(
        core_axis_name="c", subcore_axis_name="s", num_cores=1)
    sc_hist = functools.partial(
        pl.kernel,
        out_type=jax.ShapeDtypeStruct((16,), jnp.float32),
        mesh=hist_mesh,
        compiler_params=pltpu.CompilerParams(needs_layout_passes=False),
        scratch_types=[
            pltpu.VMEM((n // _NS,), jnp.float32),
            pltpu.VMEM((n // _NS,), jnp.float32),
            pltpu.VMEM((32,), jnp.float32),
            pltpu.VMEM((_NS * 32,), jnp.float32),
            pltpu.VMEM((16,), jnp.float32),
            pltpu.VMEM_SHARED((_NS * 32,), jnp.float32),
        ],
    )(functools.partial(_sc_hist_body, n_rows=n))

    out16 = sc_hist(lse.reshape(n), xt)
    return out16[0]


# speedup vs baseline: 2.3654x; 2.3654x over previous
import functools

import jax
import jax.numpy as jnp
from jax import lax
from jax.experimental import pallas as pl
from jax.experimental.pallas import tpu as pltpu
from jax.experimental.pallas import tpu_sc as plsc

_BINS = 10
_NS = 16


def _allsum(v):
    return jnp.zeros((16,), v.dtype) + jnp.sum(v)


def _ce_half(xb, t):
    m = jnp.max(xb, axis=2)
    se = jnp.sum(jnp.exp(xb - m[:, :, None]), axis=2)
    lse = m + jnp.log(se)
    col = lax.broadcasted_iota(jnp.int32, xb.shape, 2)
    xt = jnp.sum(jnp.where(col == t[:, :, None], xb, 0.0), axis=2)
    return lse - xt


def _ce_kernel(xa_ref, xb_ref, t_ref, ce_ref):
    t = t_ref[...]
    sh = xa_ref.shape[0]
    ce_ref[pl.ds(0, sh), :] = _ce_half(xa_ref[...], t[:sh, :])
    ce_ref[pl.ds(sh, sh), :] = _ce_half(xb_ref[...], t[sh:, :])


def _sc_hist_body(ce_hbm, out_hbm, ce_v, both_v, gather_v, out_v,
                  shared_both, *, n_rows):
    sid = lax.axis_index("s")
    chunk = n_rows // _NS
    base = pl.multiple_of(sid * chunk, 8)
    pltpu.sync_copy(ce_hbm.at[pl.ds(base, chunk)], ce_v)

    zeros = jnp.zeros((16,), jnp.float32)

    def step(j, carry):
        cnts, sums = carry
        ce16 = ce_v[pl.ds(j * 16, 16)]
        g = jnp.abs(jnp.exp(-ce16) - 1.0)
        b16 = (g * (_BINS - 0.0001)).astype(jnp.int32)
        new_c = []
        new_s = []
        for b in range(_BINS):
            m = b16 == b
            new_c.append(cnts[b] + jnp.where(m, 1.0, 0.0))
            new_s.append(sums[b] + jnp.where(m, ce16, 0.0))
        return new_c, new_s

    cnts, sums = lax.fori_loop(
        0, chunk // 16, step, ([zeros] * _BINS, [zeros] * _BINS))

    lane = lax.iota(jnp.int32, 16)
    hist = zeros
    ssum = zeros
    for b in range(_BINS):
        hist = jnp.where(lane == b, _allsum(cnts[b]), hist)
        ssum = jnp.where(lane == b, _allsum(sums[b]), ssum)
    both_v[pl.ds(0, 16)] = hist
    both_v[pl.ds(16, 16)] = ssum

    pltpu.sync_copy(both_v, shared_both.at[pl.ds(pl.multiple_of(sid * 32, 8), 32)])
    plsc.subcore_barrier()

    @pl.when(sid == 0)
    def _finish():
        pltpu.sync_copy(shared_both, gather_v)
        cnt = jnp.zeros((16,), jnp.float32)
        ssum2 = jnp.zeros((16,), jnp.float32)
        for w in range(_NS):
            cnt = cnt + gather_v[pl.ds(w * 32, 16)]
            ssum2 = ssum2 + gather_v[pl.ds(w * 32 + 16, 16)]
        valid = lane < _BINS
        nonempty = _allsum(jnp.where(valid & (cnt > 0.0), 1.0, 0.0))
        gd = jnp.maximum(cnt * nonempty, 0.0001)
        loss = _allsum(jnp.where(valid, ssum2 / gd, 0.0)) * (1.0 / n_rows)
        out_v[...] = loss
        pltpu.sync_copy(out_v, out_hbm)


def kernel(x, target):
    n, c = x.shape
    block_rows = 4096
    nblocks = n // block_rows
    sb = block_rows // 128
    x3 = x.reshape(n // 128, 128, c)
    t2 = target.reshape(n // 128, 128)

    sh = sb // 2
    ce = pl.pallas_call(
        _ce_kernel,
        grid=(nblocks,),
        in_specs=[
            pl.BlockSpec((sh, 128, c), lambda i: (2 * i, 0, 0)),
            pl.BlockSpec((sh, 128, c), lambda i: (2 * i + 1, 0, 0)),
            pl.BlockSpec((sb, 128), lambda i: (i, 0)),
        ],
        out_specs=pl.BlockSpec((sb, 128), lambda i: (i, 0)),
        out_shape=jax.ShapeDtypeStruct((n // 128, 128), jnp.float32),
    )(x3, x3, t2)

    mesh = plsc.VectorSubcoreMesh(
        core_axis_name="c", subcore_axis_name="s", num_cores=1)
    sc_hist = functools.partial(
        pl.kernel,
        out_type=jax.ShapeDtypeStruct((16,), jnp.float32),
        mesh=mesh,
        compiler_params=pltpu.CompilerParams(needs_layout_passes=False),
        scratch_types=[
            pltpu.VMEM((n // _NS,), jnp.float32),
            pltpu.VMEM((32,), jnp.float32),
            pltpu.VMEM((_NS * 32,), jnp.float32),
            pltpu.VMEM((16,), jnp.float32),
            pltpu.VMEM_SHARED((_NS * 32,), jnp.float32),
        ],
    )(functools.partial(_sc_hist_body, n_rows=n))

    out16 = sc_hist(ce.reshape(n))
    return out16[0]
